# Initial kernel scaffold; baseline (speedup 1.0000x reference)
import jax
import jax.numpy as jnp
from jax.experimental import pallas as pl


def kernel(score_mat, ref_knn_masks, src_knn_masks):
    raise NotImplementedError("write your pallas kernel here")


# speedup vs baseline: 8.7714x; 8.7714x over previous
import dataclasses
import functools

import jax
import jax.numpy as jnp
from jax import lax
from jax.experimental import pallas as pl
from jax.experimental.pallas import tpu as pltpu
from jax.experimental.pallas import tpu_sc as plsc

_B, _N, _M = 512, 128, 128
_TOTAL = _B * _N * _M
_NUM_CORR = 2000
_NW = 32
_PER_W = _TOTAL // _NW
_CHUNK = 8192
_NCH = _PER_W // _CHUNK
_LANES = 16
_SIGN = -(2**31)
_POSM = 2**31 - 1


def _shr(v, k):
    return lax.shift_right_logical(v, jnp.full((_LANES,), k, jnp.int32))


def _monokey16(x16):
    bits = plsc.bitcast(x16, jnp.int32)
    mkey = jnp.where(bits < 0, bits ^ _POSM, bits)
    return mkey ^ _SIGN


def _pb1(ub, idx, s0, s1):
    return None, _shr(ub, 20)


def _pb2(ub, idx, s0, s1):
    return _shr(ub, 20) == s0, _shr(ub, 8) & jnp.int32(0xFFF)


def _pb3(ub, idx, s0, s1):
    return _shr(ub, 8) == s0, ub & jnp.int32(0xFF)


def _pb4(ub, idx, s0, s1):
    return ub == s0, _shr(idx, 11)


def _pb5(ub, idx, s0, s1):
    return (ub == s0) & (_shr(idx, 11) == s1), idx & jnp.int32(0x7FF)


def _compiler_params():
    cp = pltpu.CompilerParams()
    if "needs_layout_passes" in pltpu.CompilerParams.__dataclass_fields__:
        cp = dataclasses.replace(cp, needs_layout_passes=False)
    return cp


@functools.cache
def _make_sc_pass(nbins, pred_bin_fn):
    mesh = plsc.VectorSubcoreMesh(core_axis_name="c", subcore_axis_name="s")

    @functools.partial(
        pl.kernel,
        out_type=jax.ShapeDtypeStruct((_NW, nbins), jnp.int32),
        mesh=mesh,
        scratch_types=[
            pltpu.VMEM((_CHUNK,), jnp.float32),
            pltpu.VMEM((32,), jnp.int32),
            pltpu.VMEM((nbins * _LANES,), jnp.int32),
            pltpu.VMEM((nbins,), jnp.int32),
            pltpu.SemaphoreType.DMA,
        ],
        compiler_params=_compiler_params(),
    )
    def kern(data_hbm, state_hbm, hist_hbm, buf, state_v, hist, red, sem):
        wid = lax.axis_index("c") * 16 + lax.axis_index("s")
        base = wid * _PER_W
        pltpu.async_copy(state_hbm, state_v, sem).wait()
        s0 = state_v[pl.ds(0, _LANES)]
        s1 = state_v[pl.ds(_LANES, _LANES)]
        lanes = lax.iota(jnp.int32, _LANES)
        ones = jnp.ones((_LANES,), jnp.int32)
        zeros = jnp.zeros((_LANES,), jnp.int32)

        @pl.loop(0, nbins * _LANES, step=_LANES)
        def _(i):
            hist[pl.ds(i, _LANES)] = zeros

        @pl.loop(0, _NCH)
        def _(c):
            cbase = base + c * _CHUNK
            pltpu.async_copy(
                data_hbm.at[pl.ds(cbase, _CHUNK)], buf, sem
            ).wait()

            @pl.loop(0, _CHUNK, step=_LANES)
            def _(i):
                x = buf[pl.ds(i, _LANES)]
                ub = _monokey16(x)
                idx = (cbase + i) + lanes
                pred, bin_ = pred_bin_fn(ub, idx, s0, s1)
                addr = lanes * nbins + bin_
                if pred is None:
                    plsc.addupdate_scatter(hist, [addr], ones)
                else:
                    plsc.addupdate_scatter(hist, [addr], ones, mask=pred)

        @pl.loop(0, nbins, step=_LANES)
        def _(i):
            acc = hist[pl.ds(i, _LANES)]
            for l in range(1, _LANES):
                acc = acc + hist[pl.ds(l * nbins + i, _LANES)]
            red[pl.ds(i, _LANES)] = acc

        pltpu.sync_copy(red, hist_hbm.at[wid])

    return kern


def _sc_pass1(d, s):
    return _make_sc_pass(4096, _pb1)(d, s)


def _sc_pass2(d, s):
    return _make_sc_pass(4096, _pb2)(d, s)


def _sc_pass3(d, s):
    return _make_sc_pass(256, _pb3)(d, s)


def _sc_pass4(d, s):
    return _make_sc_pass(4096, _pb4)(d, s)


def _sc_pass5(d, s):
    return _make_sc_pass(2048, _pb5)(d, s)


def _desc_step(hsum, k):
    suffix = jnp.cumsum(hsum[::-1])[::-1]
    c = jnp.sum((suffix >= k).astype(jnp.int32)) - 1
    above = suffix[c] - hsum[c]
    return c, k - above


def _asc_step(hsum, k):
    csum = jnp.cumsum(hsum)
    c = jnp.sum((csum < k).astype(jnp.int32))
    below = csum[c] - hsum[c]
    return c, k - below


def _state(a, b):
    return jnp.concatenate(
        [jnp.full((_LANES,), a, jnp.int32), jnp.full((_LANES,), b, jnp.int32)]
    )


_BB = 8


def _mark_body(score_ref, refm_ref, srcm_ref, thr_ref, corr_ref, msk_ref):
    v = thr_ref[0]
    cut = thr_ref[1]
    g = pl.program_id(0)
    x = score_ref[...]
    bits = lax.bitcast_convert_type(x, jnp.int32)
    mkey = jnp.where(bits < 0, bits ^ _POSM, bits)
    bi = lax.broadcasted_iota(jnp.int32, (_BB, _N, _M), 0)
    ri = lax.broadcasted_iota(jnp.int32, (_BB, _N, _M), 1)
    ci = lax.broadcasted_iota(jnp.int32, (_BB, _N, _M), 2)
    flat = ((g * _BB + bi) * _N + ri) * _M + ci
    sel = (mkey > v) | ((mkey == v) & (flat <= cut))
    rm = refm_ref[...] > 0
    sm = srcm_ref[...] > 0
    corr = sel & rm & sm
    corr_ref[...] = corr
    msk_ref[...] = jnp.where(corr, x, jnp.float32(0.0))


def _mark(score_mat, refm, srcm, thr):
    grid = _B // _BB
    return pl.pallas_call(
        _mark_body,
        grid=(grid,),
        in_specs=[
            pl.BlockSpec((_BB, _N, _M), lambda g: (g, 0, 0)),
            pl.BlockSpec((_BB, _N, 1), lambda g: (g, 0, 0)),
            pl.BlockSpec((_BB, 1, _M), lambda g: (g, 0, 0)),
            pl.BlockSpec(memory_space=pltpu.SMEM),
        ],
        out_specs=[
            pl.BlockSpec((_BB, _N, _M), lambda g: (g, 0, 0)),
            pl.BlockSpec((_BB, _N, _M), lambda g: (g, 0, 0)),
        ],
        out_shape=[
            jax.ShapeDtypeStruct((_B, _N, _M), jnp.bool_),
            jax.ShapeDtypeStruct((_B, _N, _M), jnp.float32),
        ],
    )(score_mat, refm, srcm, thr)


def kernel(score_mat, ref_knn_masks, src_knn_masks):
    flat = score_mat.reshape(_TOTAL)
    k = jnp.int32(_NUM_CORR)

    h1 = _sc_pass1(flat, _state(0, 0)).sum(axis=0)
    c1, k = _desc_step(h1, k)
    h2 = _sc_pass2(flat, _state(c1, 0)).sum(axis=0)
    c2, k = _desc_step(h2, k)
    p12 = (c1 << 12) | c2
    h3 = _sc_pass3(flat, _state(p12, 0)).sum(axis=0)
    c3, k = _desc_step(h3, k)
    ukeyv = (p12 << 8) | c3
    h4 = _sc_pass4(flat, _state(ukeyv, 0)).sum(axis=0)
    c4, k = _asc_step(h4, k)
    h5 = _sc_pass5(flat, _state(ukeyv, c4)).sum(axis=0)
    c5, _ = _asc_step(h5, k)
    cutoff = (c4 << 11) | c5
    v_mkey = ukeyv ^ _SIGN

    thr = jnp.stack([v_mkey, cutoff]).astype(jnp.int32)
    refm = ref_knn_masks.astype(jnp.int32)[:, :, None]
    srcm = src_knn_masks.astype(jnp.int32)[:, None, :]
    return _mark(score_mat, refm, srcm, thr)


# speedup vs baseline: 11.1767x; 1.2742x over previous
import dataclasses
import functools

import jax
import jax.numpy as jnp
from jax import lax
from jax.experimental import pallas as pl
from jax.experimental.pallas import tpu as pltpu
from jax.experimental.pallas import tpu_sc as plsc

_B, _N, _M = 512, 128, 128
_TOTAL = _B * _N * _M
_NUM_CORR = 2000
_NW = 32
_PER_W = _TOTAL // _NW
_CHUNK = 16384
_NCH = _PER_W // _CHUNK
_LANES = 16
_UNROLL = 8
_SIGN = -(2**31)
_POSM = 2**31 - 1


def _shr(v, k):
    return lax.shift_right_logical(v, jnp.full((_LANES,), k, jnp.int32))


def _pb1(ub, idx, s0, s1):
    return None, _shr(ub, 20)


def _pb2(ub, idx, s0, s1):
    return _shr(ub, 20) == s0, _shr(ub, 8) & jnp.int32(0xFFF)


def _pb3(ub, idx, s0, s1):
    return _shr(ub, 8) == s0, ub & jnp.int32(0xFF)


def _pb4(ub, idx, s0, s1):
    return ub == s0, _shr(idx, 11)


def _pb5(ub, idx, s0, s1):
    return (ub == s0) & (_shr(idx, 11) == s1), idx & jnp.int32(0x7FF)


def _compiler_params():
    cp = pltpu.CompilerParams()
    if "needs_layout_passes" in pltpu.CompilerParams.__dataclass_fields__:
        cp = dataclasses.replace(cp, needs_layout_passes=False)
    return cp


@functools.cache
def _make_sc_pass(nbins, pred_bin_fn):
    mesh = plsc.VectorSubcoreMesh(core_axis_name="c", subcore_axis_name="s")

    @functools.partial(
        pl.kernel,
        out_type=jax.ShapeDtypeStruct((_NW, nbins), jnp.int32),
        mesh=mesh,
        scratch_types=[
            pltpu.VMEM((_CHUNK,), jnp.float32),
            pltpu.VMEM((_CHUNK,), jnp.float32),
            pltpu.VMEM((32,), jnp.int32),
            pltpu.VMEM((nbins * _LANES,), jnp.int32),
            pltpu.VMEM((nbins,), jnp.int32),
            pltpu.SemaphoreType.DMA,
            pltpu.SemaphoreType.DMA,
            pltpu.SemaphoreType.DMA,
        ],
        compiler_params=_compiler_params(),
    )
    def kern(data_hbm, state_hbm, hist_hbm, buf0, buf1, state_v, hist, red,
             sem0, sem1, sems):
        wid = lax.axis_index("c") * 16 + lax.axis_index("s")
        base = wid * _PER_W
        pltpu.async_copy(state_hbm, state_v, sems).wait()
        s0 = state_v[pl.ds(0, _LANES)]
        s1 = state_v[pl.ds(_LANES, _LANES)]
        lanes = lax.iota(jnp.int32, _LANES)
        ones = jnp.ones((_LANES,), jnp.int32)
        zeros = jnp.zeros((_LANES,), jnp.int32)
        laneoff = lanes * nbins
        c31 = jnp.full((_LANES,), 31, jnp.int32)
        csign = jnp.full((_LANES,), _SIGN, jnp.int32)

        def start(chunk, bref, sem):
            pltpu.async_copy(
                data_hbm.at[pl.ds(base + chunk * _CHUNK, _CHUNK)], bref, sem
            )

        def drain(bref, sem):
            pltpu.make_async_copy(
                data_hbm.at[pl.ds(base, _CHUNK)], bref, sem
            ).wait()

        def process(bref, cbase):
            @pl.loop(0, _CHUNK, step=_LANES * _UNROLL)
            def _(i):
                for u in range(_UNROLL):
                    off = i + u * _LANES
                    x = bref[pl.ds(off, _LANES)]
                    bits = plsc.bitcast(x, jnp.int32)
                    ub = bits ^ (lax.shift_right_arithmetic(bits, c31) | csign)
                    idx = (cbase + off) + lanes
                    pred, bin_ = pred_bin_fn(ub, idx, s0, s1)
                    addr = laneoff + bin_
                    if pred is None:
                        plsc.addupdate_scatter(hist, [addr], ones)
                    else:
                        plsc.addupdate_scatter(hist, [addr], ones, mask=pred)

        @pl.loop(0, nbins * _LANES, step=_LANES * _UNROLL)
        def _(i):
            for u in range(_UNROLL):
                hist[pl.ds(i + u * _LANES, _LANES)] = zeros

        start(0, buf0, sem0)

        @pl.loop(0, _NCH, step=2)
        def _(c):
            start(c + 1, buf1, sem1)
            drain(buf0, sem0)
            process(buf0, base + c * _CHUNK)

            @pl.when(c + 2 < _NCH)
            def _():
                start(c + 2, buf0, sem0)

            drain(buf1, sem1)
            process(buf1, base + (c + 1) * _CHUNK)

        @pl.loop(0, nbins, step=_LANES)
        def _(i):
            acc = hist[pl.ds(i, _LANES)]
            for l in range(1, _LANES):
                acc = acc + hist[pl.ds(l * nbins + i, _LANES)]
            red[pl.ds(i, _LANES)] = acc

        pltpu.sync_copy(red, hist_hbm.at[wid])

    return kern


def _sc_pass1(d, s):
    return _make_sc_pass(4096, _pb1)(d, s)


def _sc_pass2(d, s):
    return _make_sc_pass(4096, _pb2)(d, s)


def _sc_pass3(d, s):
    return _make_sc_pass(256, _pb3)(d, s)


def _sc_pass4(d, s):
    return _make_sc_pass(4096, _pb4)(d, s)


def _sc_pass5(d, s):
    return _make_sc_pass(2048, _pb5)(d, s)


def _desc_step(hsum, k):
    suffix = jnp.cumsum(hsum[::-1])[::-1]
    c = jnp.sum((suffix >= k).astype(jnp.int32)) - 1
    above = suffix[c] - hsum[c]
    return c, k - above


def _asc_step(hsum, k):
    csum = jnp.cumsum(hsum)
    c = jnp.sum((csum < k).astype(jnp.int32))
    below = csum[c] - hsum[c]
    return c, k - below


def _state(a, b):
    return jnp.concatenate(
        [jnp.full((_LANES,), a, jnp.int32), jnp.full((_LANES,), b, jnp.int32)]
    )


_BB = 8


def _mark_body(score_ref, refm_ref, srcm_ref, thr_ref, corr_ref, msk_ref):
    v = thr_ref[0]
    cut = thr_ref[1]
    g = pl.program_id(0)
    x = score_ref[...]
    bits = lax.bitcast_convert_type(x, jnp.int32)
    mkey = jnp.where(bits < 0, bits ^ _POSM, bits)
    bi = lax.broadcasted_iota(jnp.int32, (_BB, _N, _M), 0)
    ri = lax.broadcasted_iota(jnp.int32, (_BB, _N, _M), 1)
    ci = lax.broadcasted_iota(jnp.int32, (_BB, _N, _M), 2)
    flat = ((g * _BB + bi) * _N + ri) * _M + ci
    sel = (mkey > v) | ((mkey == v) & (flat <= cut))
    rm = refm_ref[...] > 0
    sm = srcm_ref[...] > 0
    corr = sel & rm & sm
    corr_ref[...] = corr
    msk_ref[...] = jnp.where(corr, x, jnp.float32(0.0))


def _mark(score_mat, refm, srcm, thr):
    grid = _B // _BB
    return pl.pallas_call(
        _mark_body,
        grid=(grid,),
        in_specs=[
            pl.BlockSpec((_BB, _N, _M), lambda g: (g, 0, 0)),
            pl.BlockSpec((_BB, _N, 1), lambda g: (g, 0, 0)),
            pl.BlockSpec((_BB, 1, _M), lambda g: (g, 0, 0)),
            pl.BlockSpec(memory_space=pltpu.SMEM),
        ],
        out_specs=[
            pl.BlockSpec((_BB, _N, _M), lambda g: (g, 0, 0)),
            pl.BlockSpec((_BB, _N, _M), lambda g: (g, 0, 0)),
        ],
        out_shape=[
            jax.ShapeDtypeStruct((_B, _N, _M), jnp.bool_),
            jax.ShapeDtypeStruct((_B, _N, _M), jnp.float32),
        ],
    )(score_mat, refm, srcm, thr)


def kernel(score_mat, ref_knn_masks, src_knn_masks):
    flat = score_mat.reshape(_TOTAL)
    k = jnp.int32(_NUM_CORR)

    h1 = _sc_pass1(flat, _state(0, 0)).sum(axis=0)
    c1, k = _desc_step(h1, k)
    h2 = _sc_pass2(flat, _state(c1, 0)).sum(axis=0)
    c2, k = _desc_step(h2, k)
    p12 = (c1 << 12) | c2
    h3 = _sc_pass3(flat, _state(p12, 0)).sum(axis=0)
    c3, k = _desc_step(h3, k)
    ukeyv = (p12 << 8) | c3
    h4 = _sc_pass4(flat, _state(ukeyv, 0)).sum(axis=0)
    c4, k = _asc_step(h4, k)
    h5 = _sc_pass5(flat, _state(ukeyv, c4)).sum(axis=0)
    c5, _ = _asc_step(h5, k)
    cutoff = (c4 << 11) | c5
    v_mkey = ukeyv ^ _SIGN

    thr = jnp.stack([v_mkey, cutoff]).astype(jnp.int32)
    refm = ref_knn_masks.astype(jnp.int32)[:, :, None]
    srcm = src_knn_masks.astype(jnp.int32)[:, None, :]
    return _mark(score_mat, refm, srcm, thr)


# speedup vs baseline: 28.9881x; 2.5936x over previous
import dataclasses
import functools

import jax
import jax.numpy as jnp
from jax import lax
from jax.experimental import pallas as pl
from jax.experimental.pallas import tpu as pltpu
from jax.experimental.pallas import tpu_sc as plsc

_B, _N, _M = 512, 128, 128
_TOTAL = _B * _N * _M
_NUM_CORR = 2000
_NW = 32
_PER_W = _TOTAL // _NW
_CHUNK = 16384
_NCH = _PER_W // _CHUNK
_LANES = 16
_UNROLL = 8
_SIGN = -(2**31)
_POSM = 2**31 - 1


def _shr(v, k):
    return lax.shift_right_logical(v, jnp.full((_LANES,), k, jnp.int32))


def _pb1(ub, idx, s0, s1):
    return None, _shr(ub, 20)


def _pb2(ub, idx, s0, s1):
    return _shr(ub, 20) == s0, _shr(ub, 8) & jnp.int32(0xFFF)


def _pb3(ub, idx, s0, s1):
    return _shr(ub, 8) == s0, ub & jnp.int32(0xFF)


def _pb4(ub, idx, s0, s1):
    return ub == s0, _shr(idx, 11)


def _pb5(ub, idx, s0, s1):
    return (ub == s0) & (_shr(idx, 11) == s1), idx & jnp.int32(0x7FF)


def _compiler_params():
    cp = pltpu.CompilerParams()
    if "needs_layout_passes" in pltpu.CompilerParams.__dataclass_fields__:
        cp = dataclasses.replace(cp, needs_layout_passes=False)
    return cp


@functools.cache
def _make_sc_pass(nbins, pred_bin_fn):
    mesh = plsc.VectorSubcoreMesh(core_axis_name="c", subcore_axis_name="s")

    @functools.partial(
        pl.kernel,
        out_type=jax.ShapeDtypeStruct((_NW, nbins * _LANES), jnp.int32),
        mesh=mesh,
        scratch_types=[
            pltpu.VMEM((_CHUNK,), jnp.float32),
            pltpu.VMEM((_CHUNK,), jnp.float32),
            pltpu.VMEM((32,), jnp.int32),
            pltpu.VMEM((nbins * _LANES,), jnp.int32),
            pltpu.SemaphoreType.DMA,
            pltpu.SemaphoreType.DMA,
            pltpu.SemaphoreType.DMA,
        ],
        compiler_params=_compiler_params(),
    )
    def kern(data_hbm, state_hbm, hist_hbm, buf0, buf1, state_v, hist,
             sem0, sem1, sems):
        wid = lax.axis_index("c") * 16 + lax.axis_index("s")
        base = wid * _PER_W
        pltpu.async_copy(state_hbm, state_v, sems).wait()
        s0 = state_v[pl.ds(0, _LANES)]
        s1 = state_v[pl.ds(_LANES, _LANES)]
        lanes = lax.iota(jnp.int32, _LANES)
        ones = jnp.ones((_LANES,), jnp.int32)
        zeros = jnp.zeros((_LANES,), jnp.int32)
        c31 = jnp.full((_LANES,), 31, jnp.int32)
        csign = jnp.full((_LANES,), _SIGN, jnp.int32)

        def start(chunk, bref, sem):
            pltpu.async_copy(
                data_hbm.at[pl.ds(base + chunk * _CHUNK, _CHUNK)], bref, sem
            )

        def drain(bref, sem):
            pltpu.make_async_copy(
                data_hbm.at[pl.ds(base, _CHUNK)], bref, sem
            ).wait()

        def process(bref, cbase):
            @pl.loop(0, _CHUNK, step=_LANES * _UNROLL)
            def _(i):
                bits = [
                    plsc.bitcast(bref[pl.ds(i + u * _LANES, _LANES)],
                                 jnp.int32)
                    for u in range(_UNROLL)
                ]
                ubs = [
                    b ^ (lax.shift_right_arithmetic(b, c31) | csign)
                    for b in bits
                ]
                pbs = [
                    pred_bin_fn(ubs[u], (cbase + i + u * _LANES) + lanes,
                                s0, s1)
                    for u in range(_UNROLL)
                ]
                addrs = [(b << 4) + lanes for _, b in pbs]
                for u in range(_UNROLL):
                    pred = pbs[u][0]
                    if pred is None:
                        plsc.addupdate_scatter(hist, [addrs[u]], ones)
                    else:
                        plsc.addupdate_scatter(hist, [addrs[u]], ones,
                                               mask=pred)

        @pl.loop(0, nbins * _LANES, step=_LANES * _UNROLL)
        def _(i):
            for u in range(_UNROLL):
                hist[pl.ds(i + u * _LANES, _LANES)] = zeros

        start(0, buf0, sem0)

        @pl.loop(0, _NCH, step=2)
        def _(c):
            start(c + 1, buf1, sem1)
            drain(buf0, sem0)
            process(buf0, base + c * _CHUNK)

            @pl.when(c + 2 < _NCH)
            def _():
                start(c + 2, buf0, sem0)

            drain(buf1, sem1)
            process(buf1, base + (c + 1) * _CHUNK)

        pltpu.sync_copy(hist, hist_hbm.at[wid])

    return kern


def _sc_pass1(d, s):
    return _make_sc_pass(4096, _pb1)(d, s)


def _sc_pass2(d, s):
    return _make_sc_pass(4096, _pb2)(d, s)


def _sc_pass3(d, s):
    return _make_sc_pass(256, _pb3)(d, s)


def _sc_pass4(d, s):
    return _make_sc_pass(4096, _pb4)(d, s)


def _sc_pass5(d, s):
    return _make_sc_pass(2048, _pb5)(d, s)


def _desc_step(hsum, k):
    suffix = jnp.cumsum(hsum[::-1])[::-1]
    c = jnp.sum((suffix >= k).astype(jnp.int32)) - 1
    above = suffix[c] - hsum[c]
    return c, k - above


def _asc_step(hsum, k):
    csum = jnp.cumsum(hsum)
    c = jnp.sum((csum < k).astype(jnp.int32))
    below = csum[c] - hsum[c]
    return c, k - below


def _state(a, b):
    return jnp.concatenate(
        [jnp.full((_LANES,), a, jnp.int32), jnp.full((_LANES,), b, jnp.int32)]
    )


_BB = 8


def _mark_body(score_ref, refm_ref, srcm_ref, thr_ref, corr_ref, msk_ref):
    v = thr_ref[0]
    cut = thr_ref[1]
    g = pl.program_id(0)
    x = score_ref[...]
    bits = lax.bitcast_convert_type(x, jnp.int32)
    mkey = jnp.where(bits < 0, bits ^ _POSM, bits)
    bi = lax.broadcasted_iota(jnp.int32, (_BB, _N, _M), 0)
    ri = lax.broadcasted_iota(jnp.int32, (_BB, _N, _M), 1)
    ci = lax.broadcasted_iota(jnp.int32, (_BB, _N, _M), 2)
    flat = ((g * _BB + bi) * _N + ri) * _M + ci
    sel = (mkey > v) | ((mkey == v) & (flat <= cut))
    rm = refm_ref[...] > 0
    sm = srcm_ref[...] > 0
    corr = sel & rm & sm
    corr_ref[...] = corr
    msk_ref[...] = jnp.where(corr, x, jnp.float32(0.0))


def _mark(score_mat, refm, srcm, thr):
    grid = _B // _BB
    return pl.pallas_call(
        _mark_body,
        grid=(grid,),
        in_specs=[
            pl.BlockSpec((_BB, _N, _M), lambda g: (g, 0, 0)),
            pl.BlockSpec((_BB, _N, 1), lambda g: (g, 0, 0)),
            pl.BlockSpec((_BB, 1, _M), lambda g: (g, 0, 0)),
            pl.BlockSpec(memory_space=pltpu.SMEM),
        ],
        out_specs=[
            pl.BlockSpec((_BB, _N, _M), lambda g: (g, 0, 0)),
            pl.BlockSpec((_BB, _N, _M), lambda g: (g, 0, 0)),
        ],
        out_shape=[
            jax.ShapeDtypeStruct((_B, _N, _M), jnp.bool_),
            jax.ShapeDtypeStruct((_B, _N, _M), jnp.float32),
        ],
    )(score_mat, refm, srcm, thr)


def kernel(score_mat, ref_knn_masks, src_knn_masks):
    flat = score_mat.reshape(_TOTAL)
    k = jnp.int32(_NUM_CORR)

    def _red(h, nbins):
        return h.reshape(_NW, nbins, _LANES).sum(axis=(0, 2))

    h1 = _red(_sc_pass1(flat, _state(0, 0)), 4096)
    c1, k = _desc_step(h1, k)
    h2 = _red(_sc_pass2(flat, _state(c1, 0)), 4096)
    c2, k = _desc_step(h2, k)
    p12 = (c1 << 12) | c2
    h3 = _red(_sc_pass3(flat, _state(p12, 0)), 256)
    c3, k = _desc_step(h3, k)
    ukeyv = (p12 << 8) | c3
    h4 = _red(_sc_pass4(flat, _state(ukeyv, 0)), 4096)
    c4, k = _asc_step(h4, k)
    h5 = _red(_sc_pass5(flat, _state(ukeyv, c4)), 2048)
    c5, _ = _asc_step(h5, k)
    cutoff = (c4 << 11) | c5
    v_mkey = ukeyv ^ _SIGN

    thr = jnp.stack([v_mkey, cutoff]).astype(jnp.int32)
    refm = ref_knn_masks.astype(jnp.int32)[:, :, None]
    srcm = src_knn_masks.astype(jnp.int32)[:, None, :]
    return _mark(score_mat, refm, srcm, thr)


# speedup vs baseline: 28.9919x; 1.0001x over previous
import dataclasses
import functools

import jax
import jax.numpy as jnp
from jax import lax
from jax.experimental import pallas as pl
from jax.experimental.pallas import tpu as pltpu
from jax.experimental.pallas import tpu_sc as plsc

_B, _N, _M = 512, 128, 128
_TOTAL = _B * _N * _M
_NUM_CORR = 2000
_NW = 32
_PER_W = _TOTAL // _NW
_CHUNK = 16384
_NCH = _PER_W // _CHUNK
_LANES = 16
_UNROLL = 8
_SIGN = -(2**31)
_POSM = 2**31 - 1


def _shr(v, k):
    return lax.shift_right_logical(v, jnp.full((_LANES,), k, jnp.int32))


def _pb1(ub, idx, s0, s1):
    return None, _shr(ub, 20)


def _pb2(ub, idx, s0, s1):
    return _shr(ub, 20) == s0, _shr(ub, 8) & jnp.int32(0xFFF)


def _pb3(ub, idx, s0, s1):
    return _shr(ub, 8) == s0, ub & jnp.int32(0xFF)


def _pb4(ub, idx, s0, s1):
    return ub == s0, _shr(idx, 11)


def _pb5(ub, idx, s0, s1):
    return (ub == s0) & (_shr(idx, 11) == s1), idx & jnp.int32(0x7FF)


def _compiler_params():
    cp = pltpu.CompilerParams()
    if "needs_layout_passes" in pltpu.CompilerParams.__dataclass_fields__:
        cp = dataclasses.replace(cp, needs_layout_passes=False)
    return cp


@functools.cache
def _make_sc_pass(nbins, pred_bin_fn):
    mesh = plsc.VectorSubcoreMesh(core_axis_name="c", subcore_axis_name="s")

    @functools.partial(
        pl.kernel,
        out_type=jax.ShapeDtypeStruct((_NW, nbins * _LANES), jnp.int32),
        mesh=mesh,
        scratch_types=[
            pltpu.VMEM((_CHUNK,), jnp.float32),
            pltpu.VMEM((_CHUNK,), jnp.float32),
            pltpu.VMEM((32,), jnp.int32),
            pltpu.VMEM((nbins * _LANES,), jnp.int32),
            pltpu.SemaphoreType.DMA,
            pltpu.SemaphoreType.DMA,
            pltpu.SemaphoreType.DMA,
        ],
        compiler_params=_compiler_params(),
    )
    def kern(data_hbm, state_hbm, hist_hbm, buf0, buf1, state_v, hist,
             sem0, sem1, sems):
        wid = lax.axis_index("c") * 16 + lax.axis_index("s")
        base = wid * _PER_W
        pltpu.async_copy(state_hbm, state_v, sems).wait()
        s0 = state_v[pl.ds(0, _LANES)]
        s1 = state_v[pl.ds(_LANES, _LANES)]
        lanes = lax.iota(jnp.int32, _LANES)
        ones = jnp.ones((_LANES,), jnp.int32)
        zeros = jnp.zeros((_LANES,), jnp.int32)
        c31 = jnp.full((_LANES,), 31, jnp.int32)
        csign = jnp.full((_LANES,), _SIGN, jnp.int32)

        def start(chunk, bref, sem):
            pltpu.async_copy(
                data_hbm.at[pl.ds(base + chunk * _CHUNK, _CHUNK)], bref, sem
            )

        def drain(bref, sem):
            pltpu.make_async_copy(
                data_hbm.at[pl.ds(base, _CHUNK)], bref, sem
            ).wait()

        def process(bref, cbase):
            @pl.loop(0, _CHUNK, step=_LANES * _UNROLL)
            def _(i):
                bits = [
                    plsc.bitcast(bref[pl.ds(i + u * _LANES, _LANES)],
                                 jnp.int32)
                    for u in range(_UNROLL)
                ]
                ubs = [
                    b ^ (lax.shift_right_arithmetic(b, c31) | csign)
                    for b in bits
                ]
                pbs = [
                    pred_bin_fn(ubs[u], (cbase + i + u * _LANES) + lanes,
                                s0, s1)
                    for u in range(_UNROLL)
                ]
                addrs = [
                    (pbs[u][1] << 4) | ((lanes + u) & 15)
                    for u in range(_UNROLL)
                ]
                for u in range(_UNROLL):
                    pred = pbs[u][0]
                    if pred is None:
                        plsc.addupdate_scatter(hist, [addrs[u]], ones)
                    else:
                        plsc.addupdate_scatter(hist, [addrs[u]], ones,
                                               mask=pred)

        @pl.loop(0, nbins * _LANES, step=_LANES * _UNROLL)
        def _(i):
            for u in range(_UNROLL):
                hist[pl.ds(i + u * _LANES, _LANES)] = zeros

        start(0, buf0, sem0)

        @pl.loop(0, _NCH, step=2)
        def _(c):
            start(c + 1, buf1, sem1)
            drain(buf0, sem0)
            process(buf0, base + c * _CHUNK)

            @pl.when(c + 2 < _NCH)
            def _():
                start(c + 2, buf0, sem0)

            drain(buf1, sem1)
            process(buf1, base + (c + 1) * _CHUNK)

        pltpu.sync_copy(hist, hist_hbm.at[wid])

    return kern


def _sc_pass1(d, s):
    return _make_sc_pass(4096, _pb1)(d, s)


def _sc_pass2(d, s):
    return _make_sc_pass(4096, _pb2)(d, s)


def _sc_pass3(d, s):
    return _make_sc_pass(256, _pb3)(d, s)


def _sc_pass4(d, s):
    return _make_sc_pass(4096, _pb4)(d, s)


def _sc_pass5(d, s):
    return _make_sc_pass(2048, _pb5)(d, s)


def _desc_step(hsum, k):
    suffix = jnp.cumsum(hsum[::-1])[::-1]
    c = jnp.sum((suffix >= k).astype(jnp.int32)) - 1
    above = suffix[c] - hsum[c]
    return c, k - above


def _asc_step(hsum, k):
    csum = jnp.cumsum(hsum)
    c = jnp.sum((csum < k).astype(jnp.int32))
    below = csum[c] - hsum[c]
    return c, k - below


def _state(a, b):
    return jnp.concatenate(
        [jnp.full((_LANES,), a, jnp.int32), jnp.full((_LANES,), b, jnp.int32)]
    )


_BB = 8


def _mark_body(score_ref, refm_ref, srcm_ref, thr_ref, corr_ref, msk_ref):
    v = thr_ref[0]
    cut = thr_ref[1]
    g = pl.program_id(0)
    x = score_ref[...]
    bits = lax.bitcast_convert_type(x, jnp.int32)
    mkey = jnp.where(bits < 0, bits ^ _POSM, bits)
    bi = lax.broadcasted_iota(jnp.int32, (_BB, _N, _M), 0)
    ri = lax.broadcasted_iota(jnp.int32, (_BB, _N, _M), 1)
    ci = lax.broadcasted_iota(jnp.int32, (_BB, _N, _M), 2)
    flat = ((g * _BB + bi) * _N + ri) * _M + ci
    sel = (mkey > v) | ((mkey == v) & (flat <= cut))
    rm = refm_ref[...] > 0
    sm = srcm_ref[...] > 0
    corr = sel & rm & sm
    corr_ref[...] = corr
    msk_ref[...] = jnp.where(corr, x, jnp.float32(0.0))


def _mark(score_mat, refm, srcm, thr):
    grid = _B // _BB
    return pl.pallas_call(
        _mark_body,
        grid=(grid,),
        in_specs=[
            pl.BlockSpec((_BB, _N, _M), lambda g: (g, 0, 0)),
            pl.BlockSpec((_BB, _N, 1), lambda g: (g, 0, 0)),
            pl.BlockSpec((_BB, 1, _M), lambda g: (g, 0, 0)),
            pl.BlockSpec(memory_space=pltpu.SMEM),
        ],
        out_specs=[
            pl.BlockSpec((_BB, _N, _M), lambda g: (g, 0, 0)),
            pl.BlockSpec((_BB, _N, _M), lambda g: (g, 0, 0)),
        ],
        out_shape=[
            jax.ShapeDtypeStruct((_B, _N, _M), jnp.bool_),
            jax.ShapeDtypeStruct((_B, _N, _M), jnp.float32),
        ],
    )(score_mat, refm, srcm, thr)


def kernel(score_mat, ref_knn_masks, src_knn_masks):
    flat = score_mat.reshape(_TOTAL)
    k = jnp.int32(_NUM_CORR)

    def _red(h, nbins):
        return h.reshape(_NW, nbins, _LANES).sum(axis=(0, 2))

    h1 = _red(_sc_pass1(flat, _state(0, 0)), 4096)
    c1, k = _desc_step(h1, k)
    h2 = _red(_sc_pass2(flat, _state(c1, 0)), 4096)
    c2, k = _desc_step(h2, k)
    p12 = (c1 << 12) | c2
    h3 = _red(_sc_pass3(flat, _state(p12, 0)), 256)
    c3, k = _desc_step(h3, k)
    ukeyv = (p12 << 8) | c3
    h4 = _red(_sc_pass4(flat, _state(ukeyv, 0)), 4096)
    c4, k = _asc_step(h4, k)
    h5 = _red(_sc_pass5(flat, _state(ukeyv, c4)), 2048)
    c5, _ = _asc_step(h5, k)
    cutoff = (c4 << 11) | c5
    v_mkey = ukeyv ^ _SIGN

    thr = jnp.stack([v_mkey, cutoff]).astype(jnp.int32)
    refm = ref_knn_masks.astype(jnp.int32)[:, :, None]
    srcm = src_knn_masks.astype(jnp.int32)[:, None, :]
    return _mark(score_mat, refm, srcm, thr)
